# Initial kernel scaffold; baseline (speedup 1.0000x reference)
import jax
import jax.numpy as jnp
from jax.experimental import pallas as pl


def kernel(boxes, scores):
    raise NotImplementedError("write your pallas kernel here")


# speedup vs baseline: 8.1085x; 8.1085x over previous
import jax
import jax.numpy as jnp
from jax.experimental import pallas as pl
from jax.experimental.pallas import tpu as pltpu

_C = 80
_PRE = 1000
_PAD = 1024
_B = 128
_NB = _PAD // _B
_SCORE_T = 0.05
_NMS_T = 0.5
_DET = 100


def _overlap_tile(x1a, y1a, x2a, y2a, aa, x1b, y1b, x2b, y2b, ab):
    ltx = jnp.maximum(x1a[:, None, :], x1b[None, :, :])
    rbx = jnp.minimum(x2a[:, None, :], x2b[None, :, :])
    lty = jnp.maximum(y1a[:, None, :], y1b[None, :, :])
    rby = jnp.minimum(y2a[:, None, :], y2b[None, :, :])
    w = jnp.clip(rbx - ltx, 0.0)
    h = jnp.clip(rby - lty, 0.0)
    inter = w * h
    union = aa[:, None, :] + ab[None, :, :] - inter + 1e-9
    return jnp.where(inter / union > _NMS_T, 1.0, 0.0)


def _nms_kernel(x1_ref, y1_ref, x2_ref, y2_ref, v_ref, keep_ref, o_ref, sup_ref):
    sup_ref[...] = jnp.zeros((_PAD, _C), jnp.float32)
    row_iota = jax.lax.broadcasted_iota(jnp.int32, (_B, 1), 0)

    def block_coords(b):
        s = b * _B
        x1 = x1_ref[s:s + _B, :]
        y1 = y1_ref[s:s + _B, :]
        x2 = x2_ref[s:s + _B, :]
        y2 = y2_ref[s:s + _B, :]
        area = (x2 - x1) * (y2 - y1)
        return x1, y1, x2, y2, area

    for b in range(_NB):
        s = b * _B
        x1, y1, x2, y2, area = block_coords(b)
        o_ref[...] = _overlap_tile(x1, y1, x2, y2, area, x1, y1, x2, y2, area)
        vf = jnp.where((v_ref[s:s + _B, :] > _SCORE_T) & (sup_ref[s:s + _B, :] < 0.5),
                       1.0, 0.0)

        def step(i, carry):
            supT, keepT = carry
            onehot = jnp.where(row_iota == i, 1.0, 0.0)
            s_i = jnp.max(supT * onehot, axis=0, keepdims=True)
            v_i = jnp.max(vf * onehot, axis=0, keepdims=True)
            k_i = v_i * (1.0 - s_i)
            orow = o_ref[pl.ds(i, 1), :, :].reshape(_B, _C)
            later = jnp.where(row_iota > i, 1.0, 0.0)
            supT = jnp.maximum(supT, k_i * orow * later)
            keepT = keepT + k_i * onehot
            return supT, keepT

        zero = jnp.zeros((_B, _C), jnp.float32)
        _, keepT = jax.lax.fori_loop(0, _B, step, (zero, zero))
        keep_ref[s:s + _B, :] = keepT

        for q in range(b + 1, _NB):
            t = q * _B
            xq1, yq1, xq2, yq2, areaq = block_coords(q)
            o = _overlap_tile(x1, y1, x2, y2, area, xq1, yq1, xq2, yq2, areaq)
            add = jnp.max(o * keepT[:, None, :], axis=0)
            sup_ref[t:t + _B, :] = jnp.maximum(sup_ref[t:t + _B, :], add)


def kernel(boxes, scores):
    cls_scores = scores[:, :_C]
    top_vals, top_idx = jax.lax.top_k(cls_scores.T, _PRE)
    cls_boxes = boxes[top_idx]

    pad = _PAD - _PRE
    vals_p = jnp.concatenate(
        [top_vals, jnp.full((_C, pad), -1.0, jnp.float32)], axis=1)
    boxes_p = jnp.concatenate(
        [cls_boxes, jnp.zeros((_C, pad, 4), jnp.float32)], axis=1)

    keepT = pl.pallas_call(
        _nms_kernel,
        out_shape=jax.ShapeDtypeStruct((_PAD, _C), jnp.float32),
        scratch_shapes=[
            pltpu.VMEM((_B, _B, _C), jnp.float32),
            pltpu.VMEM((_PAD, _C), jnp.float32),
        ],
    )(boxes_p[:, :, 0].T, boxes_p[:, :, 1].T,
      boxes_p[:, :, 2].T, boxes_p[:, :, 3].T, vals_p.T)

    keep = keepT.T[:, :_PRE] > 0.5
    flat_masked = jnp.where(keep, top_vals, -1.0).reshape(-1)
    _, flat_idx = jax.lax.top_k(flat_masked, _DET)
    final_boxes = cls_boxes.reshape(-1, 4)[flat_idx]
    final_scores = top_vals.reshape(-1)[flat_idx]
    final_classes = (flat_idx // _PRE).astype(jnp.int32)
    return final_boxes, final_scores, final_classes


# speedup vs baseline: 8.6884x; 1.0715x over previous
import jax
import jax.numpy as jnp
from jax.experimental import pallas as pl
from jax.experimental.pallas import tpu as pltpu

_C = 80
_PRE = 1000
_PAD = 1024
_B = 128
_NB = _PAD // _B
_SCORE_T = 0.05
_NMS_T = 0.5
_DET = 100


def _overlap_tile(x1a, y1a, x2a, y2a, aa, x1b, y1b, x2b, y2b, ab):
    ltx = jnp.maximum(x1a[:, None, :], x1b[None, :, :])
    rbx = jnp.minimum(x2a[:, None, :], x2b[None, :, :])
    lty = jnp.maximum(y1a[:, None, :], y1b[None, :, :])
    rby = jnp.minimum(y2a[:, None, :], y2b[None, :, :])
    w = jnp.clip(rbx - ltx, 0.0)
    h = jnp.clip(rby - lty, 0.0)
    inter = w * h
    union = aa[:, None, :] + ab[None, :, :] - inter + 1e-9
    return jnp.where(inter / union > _NMS_T, 1.0, 0.0)


def _nms_kernel(x1_ref, y1_ref, x2_ref, y2_ref, v_ref, keep_ref, o_ref, sup_ref):
    sup_ref[...] = jnp.zeros((_PAD, _C), jnp.float32)
    row_iota = jax.lax.broadcasted_iota(jnp.int32, (_B, 1), 0)

    def block_coords(b):
        s = b * _B
        x1 = x1_ref[s:s + _B, :]
        y1 = y1_ref[s:s + _B, :]
        x2 = x2_ref[s:s + _B, :]
        y2 = y2_ref[s:s + _B, :]
        area = (x2 - x1) * (y2 - y1)
        return x1, y1, x2, y2, area

    for b in range(_NB):
        s = b * _B
        x1, y1, x2, y2, area = block_coords(b)
        o_ref[...] = _overlap_tile(x1, y1, x2, y2, area, x1, y1, x2, y2, area)
        vf = jnp.where((v_ref[s:s + _B, :] > _SCORE_T) & (sup_ref[s:s + _B, :] < 0.5),
                       1.0, 0.0)

        def step(i, carry):
            supT, keepT = carry
            onehot = jnp.where(row_iota == i, 1.0, 0.0)
            s_i = jnp.max(supT * onehot, axis=0, keepdims=True)
            v_i = jnp.max(vf * onehot, axis=0, keepdims=True)
            k_i = v_i * (1.0 - s_i)
            orow = o_ref[pl.ds(i, 1), :, :].reshape(_B, _C)
            later = jnp.where(row_iota > i, 1.0, 0.0)
            supT = jnp.maximum(supT, k_i * orow * later)
            keepT = keepT + k_i * onehot
            return supT, keepT

        zero = jnp.zeros((_B, _C), jnp.float32)
        _, keepT = jax.lax.fori_loop(0, _B, step, (zero, zero))
        keep_ref[s:s + _B, :] = keepT

        for q in range(b + 1, _NB):
            t = q * _B
            xq1, yq1, xq2, yq2, areaq = block_coords(q)
            o = _overlap_tile(x1, y1, x2, y2, area, xq1, yq1, xq2, yq2, areaq)
            add = jnp.max(o * keepT[:, None, :], axis=0)
            sup_ref[t:t + _B, :] = jnp.maximum(sup_ref[t:t + _B, :], add)


def kernel(boxes, scores):
    cls_scores = scores[:, :_C]
    top_vals, top_idx = jax.lax.approx_max_k(
        cls_scores.T, _PRE, recall_target=1.0)
    cls_boxes = boxes[top_idx]

    pad = _PAD - _PRE
    vals_p = jnp.concatenate(
        [top_vals, jnp.full((_C, pad), -1.0, jnp.float32)], axis=1)
    boxes_p = jnp.concatenate(
        [cls_boxes, jnp.zeros((_C, pad, 4), jnp.float32)], axis=1)

    keepT = pl.pallas_call(
        _nms_kernel,
        out_shape=jax.ShapeDtypeStruct((_PAD, _C), jnp.float32),
        scratch_shapes=[
            pltpu.VMEM((_B, _B, _C), jnp.float32),
            pltpu.VMEM((_PAD, _C), jnp.float32),
        ],
    )(boxes_p[:, :, 0].T, boxes_p[:, :, 1].T,
      boxes_p[:, :, 2].T, boxes_p[:, :, 3].T, vals_p.T)

    keep = keepT.T[:, :_PRE] > 0.5
    flat_masked = jnp.where(keep, top_vals, -1.0).reshape(-1)
    _, flat_idx = jax.lax.top_k(flat_masked, _DET)
    final_boxes = cls_boxes.reshape(-1, 4)[flat_idx]
    final_scores = top_vals.reshape(-1)[flat_idx]
    final_classes = (flat_idx // _PRE).astype(jnp.int32)
    return final_boxes, final_scores, final_classes


# speedup vs baseline: 8.9177x; 1.0264x over previous
import jax
import jax.numpy as jnp
from jax.experimental import pallas as pl
from jax.experimental.pallas import tpu as pltpu

_C = 80
_PRE = 1000
_PAD = 1024
_B = 128
_NB = _PAD // _B
_SCORE_T = 0.05
_NMS_T = 0.5
_DET = 100


def _overlap_tile(x1a, y1a, x2a, y2a, aa, x1b, y1b, x2b, y2b, ab):
    ltx = jnp.maximum(x1a[:, None, :], x1b[None, :, :])
    rbx = jnp.minimum(x2a[:, None, :], x2b[None, :, :])
    lty = jnp.maximum(y1a[:, None, :], y1b[None, :, :])
    rby = jnp.minimum(y2a[:, None, :], y2b[None, :, :])
    w = jnp.clip(rbx - ltx, 0.0)
    h = jnp.clip(rby - lty, 0.0)
    inter = w * h
    union = aa[:, None, :] + ab[None, :, :] - inter + 1e-9
    return jnp.where(inter / union > _NMS_T, 1.0, 0.0)


def _nms_kernel(x1_ref, y1_ref, x2_ref, y2_ref, v_ref,
                out_ref, o_ref, sup_ref, vf_ref, sup2_ref, keep_ref):
    sup_ref[...] = jnp.zeros((_PAD, _C), jnp.float32)
    row_iota = jax.lax.broadcasted_iota(jnp.int32, (_B, 1), 0)

    def block_coords(b):
        s = b * _B
        x1 = x1_ref[s:s + _B, :]
        y1 = y1_ref[s:s + _B, :]
        x2 = x2_ref[s:s + _B, :]
        y2 = y2_ref[s:s + _B, :]
        area = (x2 - x1) * (y2 - y1)
        return x1, y1, x2, y2, area

    for b in range(_NB):
        s = b * _B
        x1, y1, x2, y2, area = block_coords(b)
        o_ref[...] = _overlap_tile(x1, y1, x2, y2, area, x1, y1, x2, y2, area)
        vf_ref[...] = jnp.where(
            (v_ref[s:s + _B, :] > _SCORE_T) & (sup_ref[s:s + _B, :] < 0.5),
            1.0, 0.0)
        sup2_ref[...] = jnp.zeros((_B, _C), jnp.float32)

        def step(t, carry):
            i = 2 * t
            v0 = vf_ref[pl.ds(i, 1), :]
            s0 = sup2_ref[pl.ds(i, 1), :]
            k0 = v0 * (1.0 - s0)
            orow0 = o_ref[pl.ds(i, 1), :, :].reshape(_B, _C)
            orow1 = o_ref[pl.ds(i + 1, 1), :, :].reshape(_B, _C)
            o01 = o_ref[pl.ds(i, 1), pl.ds(i + 1, 1), :].reshape(1, _C)
            v1 = vf_ref[pl.ds(i + 1, 1), :]
            s1 = sup2_ref[pl.ds(i + 1, 1), :]
            k1 = v1 * (1.0 - s1) * (1.0 - k0 * o01)
            later0 = jnp.where(row_iota > i, 1.0, 0.0)
            later1 = jnp.where(row_iota > i + 1, 1.0, 0.0)
            sup2_ref[...] = jnp.maximum(sup2_ref[...],
                                        jnp.maximum(k0 * orow0 * later0,
                                                    k1 * orow1 * later1))
            keep_ref[pl.ds(i, 1), :] = k0
            keep_ref[pl.ds(i + 1, 1), :] = k1
            return carry

        jax.lax.fori_loop(0, _B // 2, step, 0)
        keepb = keep_ref[...]
        out_ref[s:s + _B, :] = jnp.where(keepb > 0.5, v_ref[s:s + _B, :], -1.0)

        for q in range(b + 1, _NB):
            t = q * _B
            xq1, yq1, xq2, yq2, areaq = block_coords(q)
            o = _overlap_tile(x1, y1, x2, y2, area, xq1, yq1, xq2, yq2, areaq)
            add = jnp.max(o * keepb[:, None, :], axis=0)
            sup_ref[t:t + _B, :] = jnp.maximum(sup_ref[t:t + _B, :], add)


def kernel(boxes, scores):
    cls_scores = scores[:, :_C]
    top_vals, top_idx = jax.lax.approx_max_k(
        cls_scores.T, _PRE, recall_target=1.0)
    cls_boxes = boxes[top_idx]

    pad = _PAD - _PRE
    vals_p = jnp.concatenate(
        [top_vals, jnp.full((_C, pad), -1.0, jnp.float32)], axis=1)
    boxes_p = jnp.concatenate(
        [cls_boxes, jnp.zeros((_C, pad, 4), jnp.float32)], axis=1)

    maskedT = pl.pallas_call(
        _nms_kernel,
        out_shape=jax.ShapeDtypeStruct((_PAD, _C), jnp.float32),
        scratch_shapes=[
            pltpu.VMEM((_B, _B, _C), jnp.float32),
            pltpu.VMEM((_PAD, _C), jnp.float32),
            pltpu.VMEM((_B, _C), jnp.float32),
            pltpu.VMEM((_B, _C), jnp.float32),
            pltpu.VMEM((_B, _C), jnp.float32),
        ],
    )(boxes_p[:, :, 0].T, boxes_p[:, :, 1].T,
      boxes_p[:, :, 2].T, boxes_p[:, :, 3].T, vals_p.T)

    pos_iota = jnp.arange(_PAD, dtype=jnp.int32)[None, :]
    flat_masked = jnp.where(pos_iota >= _PRE, -2.0, maskedT.T).reshape(-1)
    _, flat_idx = jax.lax.top_k(flat_masked, _DET)
    final_boxes = boxes_p.reshape(-1, 4)[flat_idx]
    final_scores = vals_p.reshape(-1)[flat_idx]
    final_classes = (flat_idx // _PAD).astype(jnp.int32)
    return final_boxes, final_scores, final_classes
